# baseline (speedup 1.0000x reference)
import functools

import jax
import jax.numpy as jnp
from jax import lax
from jax.experimental import pallas as pl
from jax.experimental.pallas import tpu as pltpu
from jax.experimental.pallas import tpu_sc as plsc


def _make_sc_kernel(N, C, K, NC, NS, L):
    NW = NC * NS
    rows_per_w = N // NW
    G = K // L
    R = 64
    NCHUNK = rows_per_w // R
    U = 4

    mesh = plsc.VectorSubcoreMesh(core_axis_name="c", subcore_axis_name="s")

    @functools.partial(
        pl.kernel,
        mesh=mesh,
        out_type=jax.ShapeDtypeStruct((N * K,), jnp.float32),
        compiler_params=pltpu.CompilerParams(needs_layout_passes=False),
        scratch_types=[
            pltpu.VMEM((R * C,), jnp.float32),
            pltpu.VMEM((R * C,), jnp.float32),
            pltpu.VMEM((R * K,), jnp.float32),
            pltpu.VMEM((R * K,), jnp.float32),
            pltpu.VMEM((K,), jnp.int32),
            pltpu.VMEM((K,), jnp.float32),
            pltpu.VMEM((K,), jnp.float32),
            pltpu.SemaphoreType.DMA,
            pltpu.SemaphoreType.DMA,
            pltpu.SemaphoreType.DMA,
            pltpu.SemaphoreType.DMA,
        ],
    )
    def sc_kernel(y_hbm, w_hbm, b_hbm, onn_hbm, out_hbm,
                  in0, in1, out0, out1, onn_v, w_v, b_v,
                  sem_in0, sem_in1, sem_out0, sem_out1):
        wid = lax.axis_index("s") * NC + lax.axis_index("c")
        pltpu.sync_copy(onn_hbm, onn_v)
        pltpu.sync_copy(w_hbm, w_v)
        pltpu.sync_copy(b_hbm, b_v)

        base = wid * rows_per_w
        inbufs = [in0, in1]
        outbufs = [out0, out1]
        sin = [sem_in0, sem_in1]
        sout = [sem_out0, sem_out1]

        in_copies = [
            pltpu.make_async_copy(
                y_hbm.at[pl.ds((base + c * R) * C, R * C)],
                inbufs[c % 2], sin[c % 2])
            for c in range(NCHUNK)
        ]
        out_copies = [
            pltpu.make_async_copy(
                outbufs[c % 2],
                out_hbm.at[pl.ds((base + c * R) * K, R * K)],
                sout[c % 2])
            for c in range(NCHUNK)
        ]

        onn_g = [onn_v[pl.ds(g * L, L)] for g in range(G)]
        w_g = [w_v[pl.ds(g * L, L)] for g in range(G)]
        b_g = [b_v[pl.ds(g * L, L)] for g in range(G)]

        in_copies[0].start()
        for c in range(NCHUNK):
            if c + 1 < NCHUNK:
                in_copies[c + 1].start()
            in_copies[c].wait()
            if c >= 2:
                out_copies[c - 2].wait()

            inbuf = inbufs[c % 2]
            outbuf = outbufs[c % 2]

            def body(i, carry, inbuf=inbuf, outbuf=outbuf):
                r0 = i * U
                for u in range(U):
                    rb = (r0 + u) * C
                    ob = (r0 + u) * K
                    for g in range(G):
                        idx = onn_g[g] + rb
                        v = plsc.load_gather(inbuf, [idx])
                        outbuf[pl.ds(ob + g * L, L)] = v * w_g[g] + b_g[g]
                return carry

            lax.fori_loop(0, R // U, body, 0)
            out_copies[c].start()

        out_copies[NCHUNK - 2].wait()
        out_copies[NCHUNK - 1].wait()

    return sc_kernel


def kernel(Y_full, weights, bias, output_node_order):
    N, C = Y_full.shape
    K = output_node_order.shape[0]
    info = plsc.get_sparse_core_info()
    NC, NS, L = info.num_cores, info.num_subcores, info.num_lanes

    sc_kernel = _make_sc_kernel(N, C, K, NC, NS, L)
    out_flat = sc_kernel(
        Y_full.reshape(-1),
        weights,
        bias,
        output_node_order.astype(jnp.int32),
    )
    return out_flat.reshape(N, K)


# speedup vs baseline: 1.2261x; 1.2261x over previous
import functools

import jax
import jax.numpy as jnp
from jax import lax
from jax.experimental import pallas as pl
from jax.experimental.pallas import tpu as pltpu
from jax.experimental.pallas import tpu_sc as plsc


def _make_sc_kernel(N, C, K, NC, NS, L):
    NW = NC * NS
    rows_per_w = N // NW
    G = K // L
    R = 64
    NCHUNK = rows_per_w // R
    U = 4

    mesh = plsc.VectorSubcoreMesh(core_axis_name="c", subcore_axis_name="s")

    @functools.partial(
        pl.kernel,
        mesh=mesh,
        out_type=jax.ShapeDtypeStruct((N * K,), jnp.float32),
        compiler_params=pltpu.CompilerParams(needs_layout_passes=False),
        scratch_types=[
            pltpu.VMEM((R * C,), jnp.float32),
            pltpu.VMEM((R * C,), jnp.float32),
            pltpu.VMEM((R * K,), jnp.float32),
            pltpu.VMEM((R * K,), jnp.float32),
            pltpu.VMEM((K,), jnp.int32),
            pltpu.VMEM((K,), jnp.float32),
            pltpu.VMEM((K,), jnp.float32),
            pltpu.SemaphoreType.DMA,
            pltpu.SemaphoreType.DMA,
            pltpu.SemaphoreType.DMA,
            pltpu.SemaphoreType.DMA,
        ],
    )
    def sc_kernel(y_hbm, w_hbm, b_hbm, onn_hbm, out_hbm,
                  in0, in1, out0, out1, onn_v, w_v, b_v,
                  sem_in0, sem_in1, sem_out0, sem_out1):
        wid = lax.axis_index("s") * NC + lax.axis_index("c")
        pltpu.sync_copy(onn_hbm, onn_v)
        pltpu.sync_copy(w_hbm, w_v)
        pltpu.sync_copy(b_hbm, b_v)

        base = wid * rows_per_w
        inbufs = [in0, in1]
        outbufs = [out0, out1]
        sin = [sem_in0, sem_in1]
        sout = [sem_out0, sem_out1]

        in_copies = [
            pltpu.make_async_copy(
                y_hbm.at[pl.ds((base + c * R) * C, R * C)],
                inbufs[c % 2], sin[c % 2])
            for c in range(NCHUNK)
        ]
        out_copies = [
            pltpu.make_async_copy(
                outbufs[c % 2],
                out_hbm.at[pl.ds((base + c * R) * K, R * K)],
                sout[c % 2])
            for c in range(NCHUNK)
        ]

        onn_g = [onn_v[pl.ds(g * L, L)] for g in range(G)]
        w_g = [w_v[pl.ds(g * L, L)] for g in range(G)]
        b_g = [b_v[pl.ds(g * L, L)] for g in range(G)]

        in_copies[0].start()
        for c in range(NCHUNK):
            if c + 1 < NCHUNK:
                in_copies[c + 1].start()
            in_copies[c].wait()
            if c >= 2:
                out_copies[c - 2].wait()

            inbuf = inbufs[c % 2]
            outbuf = outbufs[c % 2]

            @plsc.parallel_loop(0, R, unroll=U)
            def body(r, inbuf=inbuf, outbuf=outbuf):
                rb = r * C
                ob = r * K
                for g in range(G):
                    idx = onn_g[g] + rb
                    v = plsc.load_gather(inbuf, [idx])
                    outbuf[pl.ds(ob + g * L, L)] = v * w_g[g] + b_g[g]
            out_copies[c].start()

        out_copies[NCHUNK - 2].wait()
        out_copies[NCHUNK - 1].wait()

    return sc_kernel


def kernel(Y_full, weights, bias, output_node_order):
    N, C = Y_full.shape
    K = output_node_order.shape[0]
    info = plsc.get_sparse_core_info()
    NC, NS, L = info.num_cores, info.num_subcores, info.num_lanes

    sc_kernel = _make_sc_kernel(N, C, K, NC, NS, L)
    out_flat = sc_kernel(
        Y_full.reshape(-1),
        weights,
        bias,
        output_node_order.astype(jnp.int32),
    )
    return out_flat.reshape(N, K)


# speedup vs baseline: 1.4291x; 1.1655x over previous
import functools

import jax
import jax.numpy as jnp
from jax import lax
from jax.experimental import pallas as pl
from jax.experimental.pallas import tpu as pltpu
from jax.experimental.pallas import tpu_sc as plsc


def _make_sc_kernel(N, C, K, NC, NS, L):
    NW = NC * NS
    rows_per_w = N // NW
    G = K // L
    R = 64
    NCHUNK = rows_per_w // R
    U = 4

    mesh = plsc.VectorSubcoreMesh(core_axis_name="c", subcore_axis_name="s")

    @functools.partial(
        pl.kernel,
        mesh=mesh,
        out_type=jax.ShapeDtypeStruct((N, K), jnp.float32),
        compiler_params=pltpu.CompilerParams(needs_layout_passes=False),
        scratch_types=[
            pltpu.VMEM((R, C), jnp.float32),
            pltpu.VMEM((R, C), jnp.float32),
            pltpu.VMEM((R, K), jnp.float32),
            pltpu.VMEM((R, K), jnp.float32),
            pltpu.VMEM((K,), jnp.int32),
            pltpu.VMEM((K,), jnp.float32),
            pltpu.VMEM((K,), jnp.float32),
            pltpu.SemaphoreType.DMA,
            pltpu.SemaphoreType.DMA,
            pltpu.SemaphoreType.DMA,
            pltpu.SemaphoreType.DMA,
        ],
    )
    def sc_kernel(y_hbm, w_hbm, b_hbm, onn_hbm, out_hbm,
                  in0, in1, out0, out1, onn_v, w_v, b_v,
                  sem_in0, sem_in1, sem_out0, sem_out1):
        wid = lax.axis_index("s") * NC + lax.axis_index("c")
        pltpu.sync_copy(onn_hbm, onn_v)
        pltpu.sync_copy(w_hbm, w_v)
        pltpu.sync_copy(b_hbm, b_v)

        base = wid * rows_per_w
        inbufs = [in0, in1]
        outbufs = [out0, out1]
        sin = [sem_in0, sem_in1]
        sout = [sem_out0, sem_out1]

        in_copies = [
            pltpu.make_async_copy(
                y_hbm.at[pl.ds(base + c * R, R)],
                inbufs[c % 2], sin[c % 2])
            for c in range(NCHUNK)
        ]
        out_copies = [
            pltpu.make_async_copy(
                outbufs[c % 2],
                out_hbm.at[pl.ds(base + c * R, R)],
                sout[c % 2])
            for c in range(NCHUNK)
        ]

        onn_g = [onn_v[pl.ds(g * L, L)] for g in range(G)]
        w_g = [w_v[pl.ds(g * L, L)] for g in range(G)]
        b_g = [b_v[pl.ds(g * L, L)] for g in range(G)]

        in_copies[0].start()
        for c in range(NCHUNK):
            if c + 1 < NCHUNK:
                in_copies[c + 1].start()
            in_copies[c].wait()
            if c >= 2:
                out_copies[c - 2].wait()

            inbuf = inbufs[c % 2]
            outbuf = outbufs[c % 2]

            @plsc.parallel_loop(0, R, unroll=U)
            def body(r, inbuf=inbuf, outbuf=outbuf):
                row = jnp.full((L,), r, dtype=jnp.int32)
                for g in range(G):
                    v = plsc.load_gather(inbuf, [row, onn_g[g]])
                    outbuf[r, pl.ds(g * L, L)] = v * w_g[g] + b_g[g]

            out_copies[c].start()

        out_copies[NCHUNK - 2].wait()
        out_copies[NCHUNK - 1].wait()

    return sc_kernel


def kernel(Y_full, weights, bias, output_node_order):
    N, C = Y_full.shape
    K = output_node_order.shape[0]
    info = plsc.get_sparse_core_info()
    NC, NS, L = info.num_cores, info.num_subcores, info.num_lanes

    sc_kernel = _make_sc_kernel(N, C, K, NC, NS, L)
    return sc_kernel(Y_full, weights, bias, output_node_order.astype(jnp.int32))
